# Initial kernel scaffold; baseline (speedup 1.0000x reference)
import jax
import jax.numpy as jnp
from jax.experimental import pallas as pl


def kernel(p1, p2, p5, p3, p4, p6, W1, b1, E2, E3, E4, W5, b5, W6, b6):
    raise NotImplementedError("write your pallas kernel here")


# speedup vs baseline: 1.7979x; 1.7979x over previous
import functools

import jax
import jax.numpy as jnp
from jax import lax
from jax.experimental import pallas as pl
from jax.experimental.pallas import tpu as pltpu
from jax.experimental.pallas import tpu_sc as plsc

B = 16384
D4 = 16
NC, NS = 2, 16
NW = NC * NS
BPW = B // NW
CHUNK = 128
NCHUNK = BPW // CHUNK

BB = 2048


def _sc_gather(table, idx3):
    mesh = plsc.VectorSubcoreMesh(core_axis_name="c", subcore_axis_name="s")

    @functools.partial(
        pl.kernel,
        out_type=jax.ShapeDtypeStruct((B, D4), jnp.float32),
        mesh=mesh,
        scratch_types=[
            pltpu.VMEM((NCHUNK, CHUNK), jnp.int32),
            pltpu.VMEM((BPW, D4), jnp.float32),
            pltpu.SemaphoreType.DMA,
        ],
        compiler_params=pltpu.CompilerParams(use_tc_tiling_on_sc=False),
    )
    def gather_kernel(table_hbm, idx_hbm, out_hbm, idx_v, rows_v, sem):
        wid = lax.axis_index("s") * NC + lax.axis_index("c")
        pltpu.sync_copy(idx_hbm.at[wid], idx_v)
        copies = [
            pltpu.async_copy(
                table_hbm.at[idx_v.at[j]],
                rows_v.at[pl.ds(j * CHUNK, CHUNK)],
                sem,
            )
            for j in range(NCHUNK)
        ]
        for c in copies:
            c.wait()
        pltpu.sync_copy(rows_v, out_hbm.at[pl.ds(wid * BPW, BPW)])

    return gather_kernel(table, idx3)


def _dense_body(p1_ref, p2_ref, p5_ref, p3_ref, p4_ref, v4_ref,
                W1_ref, b1_ref, E2_ref, E3_ref,
                W5A_ref, W5B_ref, W5C_ref, b5_ref,
                W6A_ref, W6B_ref, b6_ref, out_ref):
    f32 = jnp.float32
    W1 = W1_ref[...]
    v1 = jnp.tanh(p1_ref[...] * W1[0:1, :] + p2_ref[...] * W1[1:2, :]
                  + p5_ref[...] * W1[2:3, :] + b1_ref[...])
    T2 = jnp.dot(E2_ref[...], W5B_ref[...], preferred_element_type=f32)
    T3 = jnp.dot(E3_ref[...], W5C_ref[...], preferred_element_type=f32)
    TT = jnp.concatenate([T2, T3], axis=0)
    lane = lax.broadcasted_iota(jnp.int32, (BB, 8), 1)
    p34 = jnp.where(lane < 4, p3_ref[...], p4_ref[...] + 4)
    oh = (p34 == lane).astype(f32)
    sel = jnp.dot(oh, TT, preferred_element_type=f32)
    v5 = jnp.tanh(jnp.dot(v1, W5A_ref[...], preferred_element_type=f32)
                  + sel + b5_ref[...])
    x6 = (jnp.dot(v4_ref[...], W6A_ref[...], preferred_element_type=f32)
          + jnp.dot(v5, W6B_ref[...], preferred_element_type=f32))
    out_ref[...] = jnp.tanh(x6 + b6_ref[...])


def _tc_dense(p1, p2, p5, p3c, p4c, v4, W1, b1r, E2, E3,
              W5A, W5B, W5C, b5r, W6A, W6B, b6r):
    grid = (B // BB,)
    col = lambda i: (i, 0)
    rep = lambda i: (0, 0)
    return pl.pallas_call(
        _dense_body,
        grid=grid,
        in_specs=[
            pl.BlockSpec((BB, 1), col),
            pl.BlockSpec((BB, 1), col),
            pl.BlockSpec((BB, 1), col),
            pl.BlockSpec((BB, 1), col),
            pl.BlockSpec((BB, 1), col),
            pl.BlockSpec((BB, D4), col),
            pl.BlockSpec((3, 8), rep),
            pl.BlockSpec((1, 8), rep),
            pl.BlockSpec((4, 8), rep),
            pl.BlockSpec((4, 8), rep),
            pl.BlockSpec((8, 64), rep),
            pl.BlockSpec((8, 64), rep),
            pl.BlockSpec((8, 64), rep),
            pl.BlockSpec((1, 64), rep),
            pl.BlockSpec((16, 128), rep),
            pl.BlockSpec((64, 128), rep),
            pl.BlockSpec((1, 128), rep),
        ],
        out_specs=pl.BlockSpec((BB, 128), col),
        out_shape=jax.ShapeDtypeStruct((B, 128), jnp.float32),
        compiler_params=pltpu.CompilerParams(
            dimension_semantics=("arbitrary",)),
    )(p1, p2, p5, p3c, p4c, v4, W1, b1r, E2, E3,
      W5A, W5B, W5C, b5r, W6A, W6B, b6r)


def kernel(p1, p2, p5, p3, p4, p6, W1, b1, E2, E3, E4, W5, b5, W6, b6):
    idx3 = p6.reshape(NW, NCHUNK, CHUNK)
    v4 = _sc_gather(E4, idx3)
    p3c = p3.reshape(B, 1)
    p4c = p4.reshape(B, 1)
    return _tc_dense(
        p1, p2, p5, p3c, p4c, v4,
        W1, b1.reshape(1, 8), E2, E3,
        W5[0:8], W5[8:16], W5[16:24], b5.reshape(1, 64),
        W6[0:16], W6[16:80], b6.reshape(1, 128),
    )


# speedup vs baseline: 2.4417x; 1.3581x over previous
import functools

import jax
import jax.numpy as jnp
from jax import lax
from jax.experimental import pallas as pl
from jax.experimental.pallas import tpu as pltpu
from jax.experimental.pallas import tpu_sc as plsc

B = 16384
D4 = 16
V4 = 352899
NT = V4 // 8
TAIL = NT * 8
NC, NS = 2, 16
NW = NC * NS
BPW = B // NW
CHUNK = 128
NCHUNK = BPW // CHUNK

BB = 2048


def _sc_gather(table, idx3):
    mesh = plsc.VectorSubcoreMesh(core_axis_name="c", subcore_axis_name="s")

    @functools.partial(
        pl.kernel,
        out_type=jax.ShapeDtypeStruct((B, D4), jnp.float32),
        mesh=mesh,
        scratch_types=[
            pltpu.VMEM((NCHUNK, CHUNK), jnp.int32),
            pltpu.VMEM((BPW, D4), jnp.float32),
            pltpu.SemaphoreType.DMA,
        ],
        compiler_params=pltpu.CompilerParams(needs_layout_passes=False),
    )
    def gather_kernel(table_hbm, idx_hbm, out_hbm, idx_v, out_v, sem):
        wid = lax.axis_index("s") * NC + lax.axis_index("c")
        pair = wid // 2
        half = wid % 2
        pltpu.sync_copy(idx_hbm.at[pair, pl.ds(half * NCHUNK, NCHUNK)], idx_v)

        for g in range(BPW // 16):
            vec = idx_v[g // 8, pl.ds((g % 8) * 16, 16)]
            for k in range(16):
                pltpu.async_copy(table_hbm.at[vec[k]],
                                 out_v.at[g * 16 + k], sem)
        pltpu.make_async_copy(
            out_hbm.at[pl.ds(wid * BPW, BPW)], out_v, sem).wait()
        pltpu.sync_copy(out_v, out_hbm.at[pl.ds(wid * BPW, BPW)])

    return gather_kernel(table, idx3)


def _dense_body(p1_ref, p2_ref, p5_ref, p3_ref, p4_ref, v4_ref,
                W1_ref, b1_ref, E2_ref, E3_ref,
                W5A_ref, W5B_ref, W5C_ref, b5_ref,
                W6A_ref, W6B_ref, b6_ref, out_ref):
    f32 = jnp.float32
    W1 = W1_ref[...]
    v1 = jnp.tanh(p1_ref[...] * W1[0:1, :] + p2_ref[...] * W1[1:2, :]
                  + p5_ref[...] * W1[2:3, :] + b1_ref[...])
    T2 = jnp.dot(E2_ref[...], W5B_ref[...], preferred_element_type=f32)
    T3 = jnp.dot(E3_ref[...], W5C_ref[...], preferred_element_type=f32)
    TT = jnp.concatenate([T2, T3], axis=0)
    lane = lax.broadcasted_iota(jnp.int32, (BB, 8), 1)
    p34 = jnp.where(lane < 4, p3_ref[...], p4_ref[...] + 4)
    oh = (p34 == lane).astype(f32)
    sel = jnp.dot(oh, TT, preferred_element_type=f32)
    v5 = jnp.tanh(jnp.dot(v1, W5A_ref[...], preferred_element_type=f32)
                  + sel + b5_ref[...])
    x6 = (jnp.dot(v4_ref[...], W6A_ref[...], preferred_element_type=f32)
          + jnp.dot(v5, W6B_ref[...], preferred_element_type=f32))
    out_ref[...] = jnp.tanh(x6 + b6_ref[...])


def _tc_dense(p1, p2, p5, p3c, p4c, v4, W1, b1r, E2, E3,
              W5A, W5B, W5C, b5r, W6A, W6B, b6r):
    grid = (B // BB,)
    col = lambda i: (i, 0)
    rep = lambda i: (0, 0)
    return pl.pallas_call(
        _dense_body,
        grid=grid,
        in_specs=[
            pl.BlockSpec((BB, 1), col),
            pl.BlockSpec((BB, 1), col),
            pl.BlockSpec((BB, 1), col),
            pl.BlockSpec((BB, 1), col),
            pl.BlockSpec((BB, 1), col),
            pl.BlockSpec((BB, D4), col),
            pl.BlockSpec((3, 8), rep),
            pl.BlockSpec((1, 8), rep),
            pl.BlockSpec((4, 8), rep),
            pl.BlockSpec((4, 8), rep),
            pl.BlockSpec((8, 64), rep),
            pl.BlockSpec((8, 64), rep),
            pl.BlockSpec((8, 64), rep),
            pl.BlockSpec((1, 64), rep),
            pl.BlockSpec((16, 128), rep),
            pl.BlockSpec((64, 128), rep),
            pl.BlockSpec((1, 128), rep),
        ],
        out_specs=pl.BlockSpec((BB, 128), col),
        out_shape=jax.ShapeDtypeStruct((B, 128), jnp.float32),
        compiler_params=pltpu.CompilerParams(
            dimension_semantics=("arbitrary",)),
    )(p1, p2, p5, p3c, p4c, v4, W1, b1r, E2, E3,
      W5A, W5B, W5C, b5r, W6A, W6B, b6r)


def kernel(p1, p2, p5, p3, p4, p6, W1, b1, E2, E3, E4, W5, b5, W6, b6):
    idx3 = p6.reshape(NW // 2, 8, CHUNK)
    v4 = _sc_gather(E4, idx3)
    p3c = p3.reshape(B, 1)
    p4c = p4.reshape(B, 1)
    return _tc_dense(
        p1, p2, p5, p3c, p4c, v4,
        W1, b1.reshape(1, 8), E2, E3,
        W5[0:8], W5[8:16], W5[16:24], b5.reshape(1, 64),
        W6[0:16], W6[16:80], b6.reshape(1, 128),
    )


# speedup vs baseline: 2.4518x; 1.0041x over previous
import functools

import jax
import jax.numpy as jnp
from jax import lax
from jax.experimental import pallas as pl
from jax.experimental.pallas import tpu as pltpu
from jax.experimental.pallas import tpu_sc as plsc

B = 16384
D4 = 16
V4 = 352899
NT = V4 // 8
TAIL = NT * 8
NC, NS = 2, 16
NW = NC * NS
BPW = B // NW
CHUNK = 128
NCHUNK = BPW // CHUNK

BB = 2048


def _sc_gather(table, idx3):
    mesh = plsc.VectorSubcoreMesh(core_axis_name="c", subcore_axis_name="s")

    @functools.partial(
        pl.kernel,
        out_type=jax.ShapeDtypeStruct((B, D4), jnp.float32),
        mesh=mesh,
        scratch_types=[
            pltpu.VMEM((NCHUNK, CHUNK), jnp.int32),
            pltpu.VMEM((BPW, D4), jnp.float32),
            pltpu.SemaphoreType.DMA,
        ],
        compiler_params=pltpu.CompilerParams(
            needs_layout_passes=False, use_tc_tiling_on_sc=True),
    )
    def gather_kernel(table_hbm, idx_hbm, out_hbm, idx_v, out_v, sem):
        wid = lax.axis_index("s") * NC + lax.axis_index("c")
        pair = wid // 2
        half = wid % 2
        pltpu.sync_copy(idx_hbm.at[pair, pl.ds(half * NCHUNK, NCHUNK)], idx_v)

        for g in range(BPW // 16):
            vec = idx_v[g // 8, pl.ds((g % 8) * 16, 16)]
            for k in range(16):
                pltpu.async_copy(table_hbm.at[vec[k]],
                                 out_v.at[g * 16 + k], sem)
        pltpu.make_async_copy(
            out_hbm.at[pl.ds(wid * BPW, BPW)], out_v, sem).wait()
        pltpu.sync_copy(out_v, out_hbm.at[pl.ds(wid * BPW, BPW)])

    return gather_kernel(table, idx3)


def _dense_body(p1_ref, p2_ref, p5_ref, p3_ref, p4_ref, v4_ref,
                W1_ref, b1_ref, E2_ref, E3_ref,
                W5A_ref, W5B_ref, W5C_ref, b5_ref,
                W6A_ref, W6B_ref, b6_ref, out_ref):
    f32 = jnp.float32
    W1 = W1_ref[...]
    v1 = jnp.tanh(p1_ref[...] * W1[0:1, :] + p2_ref[...] * W1[1:2, :]
                  + p5_ref[...] * W1[2:3, :] + b1_ref[...])
    T2 = jnp.dot(E2_ref[...], W5B_ref[...], preferred_element_type=f32)
    T3 = jnp.dot(E3_ref[...], W5C_ref[...], preferred_element_type=f32)
    TT = jnp.concatenate([T2, T3], axis=0)
    lane = lax.broadcasted_iota(jnp.int32, (BB, 8), 1)
    p34 = jnp.where(lane < 4, p3_ref[...], p4_ref[...] + 4)
    oh = (p34 == lane).astype(f32)
    sel = jnp.dot(oh, TT, preferred_element_type=f32)
    v5 = jnp.tanh(jnp.dot(v1, W5A_ref[...], preferred_element_type=f32)
                  + sel + b5_ref[...])
    x6 = (jnp.dot(v4_ref[...], W6A_ref[...], preferred_element_type=f32)
          + jnp.dot(v5, W6B_ref[...], preferred_element_type=f32))
    out_ref[...] = jnp.tanh(x6 + b6_ref[...])


def _tc_dense(p1, p2, p5, p3c, p4c, v4, W1, b1r, E2, E3,
              W5A, W5B, W5C, b5r, W6A, W6B, b6r):
    grid = (B // BB,)
    col = lambda i: (i, 0)
    rep = lambda i: (0, 0)
    return pl.pallas_call(
        _dense_body,
        grid=grid,
        in_specs=[
            pl.BlockSpec((BB, 1), col),
            pl.BlockSpec((BB, 1), col),
            pl.BlockSpec((BB, 1), col),
            pl.BlockSpec((BB, 1), col),
            pl.BlockSpec((BB, 1), col),
            pl.BlockSpec((BB, D4), col),
            pl.BlockSpec((3, 8), rep),
            pl.BlockSpec((1, 8), rep),
            pl.BlockSpec((4, 8), rep),
            pl.BlockSpec((4, 8), rep),
            pl.BlockSpec((8, 64), rep),
            pl.BlockSpec((8, 64), rep),
            pl.BlockSpec((8, 64), rep),
            pl.BlockSpec((1, 64), rep),
            pl.BlockSpec((16, 128), rep),
            pl.BlockSpec((64, 128), rep),
            pl.BlockSpec((1, 128), rep),
        ],
        out_specs=pl.BlockSpec((BB, 128), col),
        out_shape=jax.ShapeDtypeStruct((B, 128), jnp.float32),
        compiler_params=pltpu.CompilerParams(
            dimension_semantics=("arbitrary",)),
    )(p1, p2, p5, p3c, p4c, v4, W1, b1r, E2, E3,
      W5A, W5B, W5C, b5r, W6A, W6B, b6r)


def kernel(p1, p2, p5, p3, p4, p6, W1, b1, E2, E3, E4, W5, b5, W6, b6):
    idx3 = p6.reshape(NW // 2, 8, CHUNK)
    v4 = _sc_gather(E4, idx3)
    p3c = p3.reshape(B, 1)
    p4c = p4.reshape(B, 1)
    return _tc_dense(
        p1, p2, p5, p3c, p4c, v4,
        W1, b1.reshape(1, 8), E2, E3,
        W5[0:8], W5[8:16], W5[16:24], b5.reshape(1, 64),
        W6[0:16], W6[16:80], b6.reshape(1, 128),
    )
